# Initial kernel scaffold; baseline (speedup 1.0000x reference)
import jax
import jax.numpy as jnp
from jax.experimental import pallas as pl


def kernel(x, edge_index, batch, W1, b1, W2, b2):
    raise NotImplementedError("write your pallas kernel here")


# speedup vs baseline: 9.3694x; 9.3694x over previous
import functools

import jax
import jax.numpy as jnp
from jax import lax
from jax.experimental import pallas as pl
from jax.experimental.pallas import tpu as pltpu
from jax.experimental.pallas import tpu_sc as plsc

N = 10000
D = 128
E = 320000
G = 64
HALF = N // 2
DUMP = HALF
ACCR = HALF + 8
NT = 16
EW = E // NT
K = 80
NC = EW // K
ST = 320
ST15 = 200
R = 1000

_mesh = plsc.VectorSubcoreMesh(core_axis_name="c", subcore_axis_name="s")


def _clamp_indices(dst_v, idx_v, c, clo):
    for j in range(K // 16):
        dv = dst_v[c, pl.ds(j * 16, 16)]
        inr = (dv >= clo) & (dv < clo + HALF)
        idx_v[pl.ds(j * 16, 16)] = jnp.where(inr, dv - clo, DUMP)


@functools.partial(
    pl.kernel,
    mesh=_mesh,
    out_type=jax.ShapeDtypeStruct((N, D), jnp.float32),
    scratch_types=[
        pltpu.VMEM((NC, K), jnp.int32),
        pltpu.VMEM((K,), jnp.int32),
        pltpu.VMEM((K, D), jnp.float32),
        pltpu.VMEM_SHARED((ACCR, D), jnp.float32),
    ],
)
def _deg(dst_hbm, out_hbm, dst_v, idx_v, ones_v, acc_s):
    cid = lax.axis_index("c")
    sid = lax.axis_index("s")
    clo = cid * HALF
    base = sid * ST
    one = jnp.ones((16,), jnp.float32)

    def fill(i, _):
        for l in range(D // 16):
            ones_v[i, pl.ds(l * 16, 16)] = one
        return 0

    def zfill(i, _):
        for l in range(D // 16):
            ones_v[i, pl.ds(l * 16, 16)] = jnp.zeros((16,), jnp.float32)
        return 0

    lax.fori_loop(0, K, zfill, 0)

    @pl.when(sid < 15)
    def _():
        for j in range(ST // K):
            pltpu.sync_copy(ones_v, acc_s.at[pl.ds(base + j * K, K)])

    @pl.when(sid == 15)
    def _():
        for j in range(ST15 // K):
            pltpu.sync_copy(ones_v, acc_s.at[pl.ds(base + j * K, K)])
        pltpu.sync_copy(ones_v.at[pl.ds(0, ST15 % K)],
                        acc_s.at[pl.ds(base + (ST15 // K) * K, ST15 % K)])

    lax.fori_loop(0, K, fill, 0)
    pltpu.sync_copy(dst_hbm.at[sid], dst_v)
    plsc.subcore_barrier()

    def step(c, _):
        _clamp_indices(dst_v, idx_v, c, clo)
        pltpu.sync_copy(ones_v, acc_s.at[idx_v], add=True)
        return 0

    lax.fori_loop(0, NC, step, 0)
    plsc.subcore_barrier()

    @pl.when(sid < 15)
    def _():
        pltpu.sync_copy(acc_s.at[pl.ds(base, ST)],
                        out_hbm.at[pl.ds(clo + base, ST)])

    @pl.when(sid == 15)
    def _():
        pltpu.sync_copy(acc_s.at[pl.ds(base, ST15)],
                        out_hbm.at[pl.ds(clo + base, ST15)])


@functools.partial(
    pl.kernel,
    mesh=_mesh,
    out_type=jax.ShapeDtypeStruct((N, D), jnp.float32),
    scratch_types=[
        pltpu.VMEM((NC, K), jnp.int32),
        pltpu.VMEM((NC, K), jnp.int32),
        pltpu.VMEM((K,), jnp.int32),
        pltpu.VMEM((K,), jnp.int32),
        pltpu.VMEM((K, D), jnp.float32),
        pltpu.VMEM_SHARED((ACCR, D), jnp.float32),
    ],
)
def _prop(hs_hbm, src_hbm, dst_hbm, out_hbm,
          src_v, dst_v, gidx_v, idx_v, rows_v, acc_s):
    cid = lax.axis_index("c")
    sid = lax.axis_index("s")
    clo = cid * HALF
    base = sid * ST

    @pl.when(sid < 15)
    def _():
        pltpu.sync_copy(hs_hbm.at[pl.ds(clo + base, ST)],
                        acc_s.at[pl.ds(base, ST)])

    @pl.when(sid == 15)
    def _():
        pltpu.sync_copy(hs_hbm.at[pl.ds(clo + base, ST15)],
                        acc_s.at[pl.ds(base, ST15)])

    pltpu.sync_copy(src_hbm.at[sid], src_v)
    pltpu.sync_copy(dst_hbm.at[sid], dst_v)
    plsc.subcore_barrier()

    def step(c, _):
        for j in range(K // 16):
            gidx_v[pl.ds(j * 16, 16)] = src_v[c, pl.ds(j * 16, 16)]
        _clamp_indices(dst_v, idx_v, c, clo)
        pltpu.sync_copy(hs_hbm.at[gidx_v], rows_v)
        pltpu.sync_copy(rows_v, acc_s.at[idx_v], add=True)
        return 0

    lax.fori_loop(0, NC, step, 0)
    plsc.subcore_barrier()

    @pl.when(sid < 15)
    def _():
        pltpu.sync_copy(acc_s.at[pl.ds(base, ST)],
                        out_hbm.at[pl.ds(clo + base, ST)])

    @pl.when(sid == 15)
    def _():
        pltpu.sync_copy(acc_s.at[pl.ds(base, ST15)],
                        out_hbm.at[pl.ds(clo + base, ST15)])


def _mm1_body(x_ref, w_ref, deg_ref, o_ref):
    dis = lax.rsqrt(1.0 + deg_ref[...])
    o_ref[...] = jnp.dot(x_ref[...], w_ref[...],
                         preferred_element_type=jnp.float32) * dis


def _mid_body(t_ref, deg_ref, b1_ref, w2_ref, o_ref):
    dis = lax.rsqrt(1.0 + deg_ref[...])
    a = jnp.maximum(t_ref[...] * dis + b1_ref[...], 0.0)
    o_ref[...] = jnp.dot(a, w2_ref[...],
                         preferred_element_type=jnp.float32) * dis


def _final_body(t_ref, deg_ref, b2_ref, bat_ref, o_ref):
    dis = lax.rsqrt(1.0 + deg_ref[...])
    h2 = t_ref[...] * dis + b2_ref[...]
    gids = lax.broadcasted_iota(jnp.int32, (R, G), 1)
    oh = (gids == bat_ref[...]).astype(jnp.float32)
    contrib = lax.dot_general(oh, h2, (((0,), (0,)), ((), ())),
                              preferred_element_type=jnp.float32)

    @pl.when(pl.program_id(0) == 0)
    def _():
        o_ref[...] = jnp.zeros_like(o_ref)

    o_ref[...] += contrib


def _mm1(x, W1, degcol):
    return pl.pallas_call(
        _mm1_body,
        grid=(N // R,),
        in_specs=[
            pl.BlockSpec((R, D), lambda i: (i, 0)),
            pl.BlockSpec((D, D), lambda i: (0, 0)),
            pl.BlockSpec((R, 1), lambda i: (i, 0)),
        ],
        out_specs=pl.BlockSpec((R, D), lambda i: (i, 0)),
        out_shape=jax.ShapeDtypeStruct((N, D), jnp.float32),
    )(x, W1, degcol)


def _mid(tmp1, degcol, b1, W2):
    return pl.pallas_call(
        _mid_body,
        grid=(N // R,),
        in_specs=[
            pl.BlockSpec((R, D), lambda i: (i, 0)),
            pl.BlockSpec((R, 1), lambda i: (i, 0)),
            pl.BlockSpec((1, D), lambda i: (0, 0)),
            pl.BlockSpec((D, D), lambda i: (0, 0)),
        ],
        out_specs=pl.BlockSpec((R, D), lambda i: (i, 0)),
        out_shape=jax.ShapeDtypeStruct((N, D), jnp.float32),
    )(tmp1, degcol, b1, W2)


def _final(tmp2, degcol, b2, bat_col):
    return pl.pallas_call(
        _final_body,
        grid=(N // R,),
        in_specs=[
            pl.BlockSpec((R, D), lambda i: (i, 0)),
            pl.BlockSpec((R, 1), lambda i: (i, 0)),
            pl.BlockSpec((1, D), lambda i: (0, 0)),
            pl.BlockSpec((R, 1), lambda i: (i, 0)),
        ],
        out_specs=pl.BlockSpec((G, D), lambda i: (0, 0)),
        out_shape=jax.ShapeDtypeStruct((G, D), jnp.float32),
        compiler_params=pltpu.CompilerParams(
            dimension_semantics=("arbitrary",)),
    )(tmp2, degcol, b2, bat_col)


def kernel(x, edge_index, batch, W1, b1, W2, b2):
    src_r = edge_index[0].reshape(NT, NC, K)
    dst_r = edge_index[1].reshape(NT, NC, K)
    degf = _deg(dst_r)
    degcol = degf[:, :1]
    h1s = _mm1(x, W1, degcol)
    tmp1 = _prop(h1s, src_r, dst_r)
    h2s = _mid(tmp1, degcol, b1.reshape(1, D), W2)
    tmp2 = _prop(h2s, src_r, dst_r)
    return _final(tmp2, degcol, b2.reshape(1, D), batch.reshape(N, 1))


# speedup vs baseline: 13.0351x; 1.3912x over previous
import functools

import jax
import jax.numpy as jnp
from jax import lax
from jax.experimental import pallas as pl
from jax.experimental.pallas import tpu as pltpu
from jax.experimental.pallas import tpu_sc as plsc

N = 10000
D = 128
E = 320000
G = 64
HALF = N // 2
DUMP = HALF
ACCR = HALF + 8
NT = 16
EW = E // NT
K = 80
NC = EW // K
ST = 320
ST15 = 200
R = 1000

_mesh = plsc.VectorSubcoreMesh(core_axis_name="c", subcore_axis_name="s")


def _clamp_indices(dst_v, idx_v, c, clo):
    for j in range(K // 16):
        dv = dst_v[c, pl.ds(j * 16, 16)]
        inr = (dv >= clo) & (dv < clo + HALF)
        idx_v[pl.ds(j * 16, 16)] = jnp.where(inr, dv - clo, DUMP)


@functools.partial(
    pl.kernel,
    mesh=_mesh,
    out_type=jax.ShapeDtypeStruct((N, D), jnp.float32),
    scratch_types=[
        pltpu.VMEM((NC, K), jnp.int32),
        pltpu.VMEM((K,), jnp.int32),
        pltpu.VMEM((K, D), jnp.float32),
        pltpu.VMEM_SHARED((ACCR, D), jnp.float32),
    ],
)
def _deg(dst_hbm, out_hbm, dst_v, idx_v, ones_v, acc_s):
    cid = lax.axis_index("c")
    sid = lax.axis_index("s")
    clo = cid * HALF
    base = sid * ST
    one = jnp.ones((16,), jnp.float32)

    def fill(i, _):
        for l in range(D // 16):
            ones_v[i, pl.ds(l * 16, 16)] = one
        return 0

    def zfill(i, _):
        for l in range(D // 16):
            ones_v[i, pl.ds(l * 16, 16)] = jnp.zeros((16,), jnp.float32)
        return 0

    lax.fori_loop(0, K, zfill, 0)

    @pl.when(sid < 15)
    def _():
        for j in range(ST // K):
            pltpu.sync_copy(ones_v, acc_s.at[pl.ds(base + j * K, K)])

    @pl.when(sid == 15)
    def _():
        for j in range(ST15 // K):
            pltpu.sync_copy(ones_v, acc_s.at[pl.ds(base + j * K, K)])
        pltpu.sync_copy(ones_v.at[pl.ds(0, ST15 % K)],
                        acc_s.at[pl.ds(base + (ST15 // K) * K, ST15 % K)])

    lax.fori_loop(0, K, fill, 0)
    pltpu.sync_copy(dst_hbm.at[sid], dst_v)
    plsc.subcore_barrier()

    def step(c, _):
        _clamp_indices(dst_v, idx_v, c, clo)
        pltpu.sync_copy(ones_v, acc_s.at[idx_v], add=True)
        return 0

    lax.fori_loop(0, NC, step, 0)
    plsc.subcore_barrier()

    @pl.when(sid < 15)
    def _():
        pltpu.sync_copy(acc_s.at[pl.ds(base, ST)],
                        out_hbm.at[pl.ds(clo + base, ST)])

    @pl.when(sid == 15)
    def _():
        pltpu.sync_copy(acc_s.at[pl.ds(base, ST15)],
                        out_hbm.at[pl.ds(clo + base, ST15)])


@functools.partial(
    pl.kernel,
    mesh=_mesh,
    out_type=jax.ShapeDtypeStruct((N, D), jnp.float32),
    scratch_types=[
        pltpu.VMEM((NC, K), jnp.int32),
        pltpu.VMEM((NC, K), jnp.int32),
        pltpu.VMEM((2, K), jnp.int32),
        pltpu.VMEM((K,), jnp.int32),
        pltpu.VMEM((2, K, D), jnp.float32),
        pltpu.SemaphoreType.DMA((2,)),
        pltpu.VMEM_SHARED((ACCR, D), jnp.float32),
    ],
)
def _prop(hs_hbm, src_hbm, dst_hbm, out_hbm,
          src_v, dst_v, gidx_v, idx_v, rows_v, sem, acc_s):
    cid = lax.axis_index("c")
    sid = lax.axis_index("s")
    clo = cid * HALF
    base = sid * ST

    @pl.when(sid < 15)
    def _():
        pltpu.sync_copy(hs_hbm.at[pl.ds(clo + base, ST)],
                        acc_s.at[pl.ds(base, ST)])

    @pl.when(sid == 15)
    def _():
        pltpu.sync_copy(hs_hbm.at[pl.ds(clo + base, ST15)],
                        acc_s.at[pl.ds(base, ST15)])

    pltpu.sync_copy(src_hbm.at[sid], src_v)
    pltpu.sync_copy(dst_hbm.at[sid], dst_v)
    plsc.subcore_barrier()

    def build_gidx(c, b):
        for j in range(K // 16):
            gidx_v[b, pl.ds(j * 16, 16)] = src_v[c, pl.ds(j * 16, 16)]

    build_gidx(0, 0)
    pltpu.async_copy(hs_hbm.at[gidx_v.at[0]], rows_v.at[0], sem.at[0])
    build_gidx(1, 1)
    pltpu.async_copy(hs_hbm.at[gidx_v.at[1]], rows_v.at[1], sem.at[1])

    def pair(p, _):
        for b in range(2):
            c = p * 2 + b
            pltpu.make_async_copy(hs_hbm.at[gidx_v.at[b]],
                                  rows_v.at[b], sem.at[b]).wait()
            _clamp_indices(dst_v, idx_v, c, clo)
            pltpu.sync_copy(rows_v.at[b], acc_s.at[idx_v], add=True)

            @pl.when(c + 2 < NC)
            def _():
                build_gidx(c + 2, b)
                pltpu.async_copy(hs_hbm.at[gidx_v.at[b]],
                                 rows_v.at[b], sem.at[b])
        return 0

    lax.fori_loop(0, NC // 2, pair, 0)
    plsc.subcore_barrier()

    @pl.when(sid < 15)
    def _():
        pltpu.sync_copy(acc_s.at[pl.ds(base, ST)],
                        out_hbm.at[pl.ds(clo + base, ST)])

    @pl.when(sid == 15)
    def _():
        pltpu.sync_copy(acc_s.at[pl.ds(base, ST15)],
                        out_hbm.at[pl.ds(clo + base, ST15)])


def _mm1_body(x_ref, w_ref, deg_ref, o_ref):
    dis = lax.rsqrt(1.0 + deg_ref[...])
    o_ref[...] = jnp.dot(x_ref[...], w_ref[...],
                         preferred_element_type=jnp.float32) * dis


def _mid_body(t_ref, deg_ref, b1_ref, w2_ref, o_ref):
    dis = lax.rsqrt(1.0 + deg_ref[...])
    a = jnp.maximum(t_ref[...] * dis + b1_ref[...], 0.0)
    o_ref[...] = jnp.dot(a, w2_ref[...],
                         preferred_element_type=jnp.float32) * dis


def _final_body(t_ref, deg_ref, b2_ref, bat_ref, o_ref):
    dis = lax.rsqrt(1.0 + deg_ref[...])
    h2 = t_ref[...] * dis + b2_ref[...]
    gids = lax.broadcasted_iota(jnp.int32, (R, G), 1)
    oh = (gids == bat_ref[...]).astype(jnp.float32)
    contrib = lax.dot_general(oh, h2, (((0,), (0,)), ((), ())),
                              preferred_element_type=jnp.float32)

    @pl.when(pl.program_id(0) == 0)
    def _():
        o_ref[...] = jnp.zeros_like(o_ref)

    o_ref[...] += contrib


def _mm1(x, W1, degcol):
    return pl.pallas_call(
        _mm1_body,
        grid=(N // R,),
        in_specs=[
            pl.BlockSpec((R, D), lambda i: (i, 0)),
            pl.BlockSpec((D, D), lambda i: (0, 0)),
            pl.BlockSpec((R, 1), lambda i: (i, 0)),
        ],
        out_specs=pl.BlockSpec((R, D), lambda i: (i, 0)),
        out_shape=jax.ShapeDtypeStruct((N, D), jnp.float32),
    )(x, W1, degcol)


def _mid(tmp1, degcol, b1, W2):
    return pl.pallas_call(
        _mid_body,
        grid=(N // R,),
        in_specs=[
            pl.BlockSpec((R, D), lambda i: (i, 0)),
            pl.BlockSpec((R, 1), lambda i: (i, 0)),
            pl.BlockSpec((1, D), lambda i: (0, 0)),
            pl.BlockSpec((D, D), lambda i: (0, 0)),
        ],
        out_specs=pl.BlockSpec((R, D), lambda i: (i, 0)),
        out_shape=jax.ShapeDtypeStruct((N, D), jnp.float32),
    )(tmp1, degcol, b1, W2)


def _final(tmp2, degcol, b2, bat_col):
    return pl.pallas_call(
        _final_body,
        grid=(N // R,),
        in_specs=[
            pl.BlockSpec((R, D), lambda i: (i, 0)),
            pl.BlockSpec((R, 1), lambda i: (i, 0)),
            pl.BlockSpec((1, D), lambda i: (0, 0)),
            pl.BlockSpec((R, 1), lambda i: (i, 0)),
        ],
        out_specs=pl.BlockSpec((G, D), lambda i: (0, 0)),
        out_shape=jax.ShapeDtypeStruct((G, D), jnp.float32),
        compiler_params=pltpu.CompilerParams(
            dimension_semantics=("arbitrary",)),
    )(tmp2, degcol, b2, bat_col)


def kernel(x, edge_index, batch, W1, b1, W2, b2):
    src_r = edge_index[0].reshape(NT, NC, K)
    dst_r = edge_index[1].reshape(NT, NC, K)
    degf = _deg(dst_r)
    degcol = degf[:, :1]
    h1s = _mm1(x, W1, degcol)
    tmp1 = _prop(h1s, src_r, dst_r)
    h2s = _mid(tmp1, degcol, b1.reshape(1, D), W2)
    tmp2 = _prop(h2s, src_r, dst_r)
    return _final(tmp2, degcol, b2.reshape(1, D), batch.reshape(N, 1))


# speedup vs baseline: 13.3295x; 1.0226x over previous
import functools

import jax
import jax.numpy as jnp
from jax import lax
from jax.experimental import pallas as pl
from jax.experimental.pallas import tpu as pltpu
from jax.experimental.pallas import tpu_sc as plsc

N = 10000
D = 128
E = 320000
G = 64
HALF = N // 2
DUMP = HALF
ACCR = HALF + 8
NT = 16
EW = E // NT
K = 80
NC = EW // K
KP = 32
NCP = EW // KP
ST = 320
ST15 = 200
R = 1000

_mesh = plsc.VectorSubcoreMesh(core_axis_name="c", subcore_axis_name="s")


def _clamp_indices(dst_v, idx_v, c, clo, k=K):
    for j in range(k // 16):
        dv = dst_v[c, pl.ds(j * 16, 16)]
        inr = (dv >= clo) & (dv < clo + HALF)
        idx_v[pl.ds(j * 16, 16)] = jnp.where(inr, dv - clo, DUMP)


@functools.partial(
    pl.kernel,
    mesh=_mesh,
    out_type=jax.ShapeDtypeStruct((N, D), jnp.float32),
    scratch_types=[
        pltpu.VMEM((NC, K), jnp.int32),
        pltpu.VMEM((K,), jnp.int32),
        pltpu.VMEM((K, D), jnp.float32),
        pltpu.VMEM_SHARED((ACCR, D), jnp.float32),
    ],
)
def _deg(dst_hbm, out_hbm, dst_v, idx_v, ones_v, acc_s):
    cid = lax.axis_index("c")
    sid = lax.axis_index("s")
    clo = cid * HALF
    base = sid * ST
    one = jnp.ones((16,), jnp.float32)

    def fill(i, _):
        for l in range(D // 16):
            ones_v[i, pl.ds(l * 16, 16)] = one
        return 0

    def zfill(i, _):
        for l in range(D // 16):
            ones_v[i, pl.ds(l * 16, 16)] = jnp.zeros((16,), jnp.float32)
        return 0

    lax.fori_loop(0, K, zfill, 0)

    @pl.when(sid < 15)
    def _():
        for j in range(ST // K):
            pltpu.sync_copy(ones_v, acc_s.at[pl.ds(base + j * K, K)])

    @pl.when(sid == 15)
    def _():
        for j in range(ST15 // K):
            pltpu.sync_copy(ones_v, acc_s.at[pl.ds(base + j * K, K)])
        pltpu.sync_copy(ones_v.at[pl.ds(0, ST15 % K)],
                        acc_s.at[pl.ds(base + (ST15 // K) * K, ST15 % K)])

    lax.fori_loop(0, K, fill, 0)
    pltpu.sync_copy(dst_hbm.at[sid], dst_v)
    plsc.subcore_barrier()

    def step(c, _):
        _clamp_indices(dst_v, idx_v, c, clo)
        pltpu.sync_copy(ones_v, acc_s.at[idx_v], add=True)
        return 0

    lax.fori_loop(0, NC, step, 0)
    plsc.subcore_barrier()

    @pl.when(sid < 15)
    def _():
        pltpu.sync_copy(acc_s.at[pl.ds(base, ST)],
                        out_hbm.at[pl.ds(clo + base, ST)])

    @pl.when(sid == 15)
    def _():
        pltpu.sync_copy(acc_s.at[pl.ds(base, ST15)],
                        out_hbm.at[pl.ds(clo + base, ST15)])


@functools.partial(
    pl.kernel,
    mesh=_mesh,
    out_type=jax.ShapeDtypeStruct((N, D), jnp.float32),
    scratch_types=[
        pltpu.VMEM((EW,), jnp.int32),
        pltpu.VMEM((EW,), jnp.int32),
        pltpu.VMEM((5, KP), jnp.int32),
        pltpu.VMEM((KP,), jnp.int32),
        pltpu.VMEM((KP,), jnp.int32),
        pltpu.VMEM((KP,), jnp.int32),
        pltpu.VMEM((KP,), jnp.int32),
        pltpu.VMEM((KP,), jnp.int32),
        pltpu.VMEM((5, KP, D), jnp.float32),
        pltpu.SemaphoreType.DMA((5,)),
        pltpu.SemaphoreType.DMA((5,)),
        pltpu.VMEM_SHARED((ACCR, D), jnp.float32),
    ],
)
def _prop(hs_hbm, src_hbm, dst_hbm, out_hbm,
          src_v, dst_v, gidx_v, ix0, ix1, ix2, ix3, ix4,
          rows_v, gsem, ssem, acc_s):
    cid = lax.axis_index("c")
    sid = lax.axis_index("s")
    clo = cid * HALF
    base = sid * ST

    @pl.when(sid < 15)
    def _():
        pltpu.sync_copy(hs_hbm.at[pl.ds(clo + base, ST)],
                        acc_s.at[pl.ds(base, ST)])

    @pl.when(sid == 15)
    def _():
        pltpu.sync_copy(hs_hbm.at[pl.ds(clo + base, ST15)],
                        acc_s.at[pl.ds(base, ST15)])

    pltpu.sync_copy(src_hbm.at[sid], src_v)
    pltpu.sync_copy(dst_hbm.at[sid], dst_v)
    plsc.subcore_barrier()

    ix = (ix0, ix1, ix2, ix3, ix4)

    def build_gidx(c, b):
        for j in range(KP // 16):
            gidx_v[b, pl.ds(j * 16, 16)] = src_v[pl.ds(c * KP + j * 16, 16)]

    def clamp_flat(c, b):
        for j in range(KP // 16):
            dv = dst_v[pl.ds(c * KP + j * 16, 16)]
            inr = (dv >= clo) & (dv < clo + HALF)
            ix[b][pl.ds(j * 16, 16)] = jnp.where(inr, dv - clo, DUMP)

    def start_gather(b):
        pltpu.async_copy(hs_hbm.at[gidx_v.at[b]], rows_v.at[b], gsem.at[b])

    def wait_gather(b):
        pltpu.make_async_copy(hs_hbm.at[gidx_v.at[b]],
                              rows_v.at[b], gsem.at[b]).wait()

    def start_scatter(b):
        pltpu.make_async_copy(rows_v.at[b], acc_s.at[ix[b]],
                              ssem.at[b]).start(add=True)

    def wait_scatter(b):
        pltpu.make_async_copy(rows_v.at[b], acc_s.at[ix[b]],
                              ssem.at[b]).wait()

    for b in range(3):
        build_gidx(jnp.int32(b), b)
        start_gather(b)

    def turn(t, b):
        bg = (b + 3) % 5

        @pl.when(t + 3 < NCP)
        def _():
            @pl.when(t >= 2)
            def _():
                wait_scatter(bg)

            build_gidx(t + 3, bg)
            start_gather(bg)

        wait_gather(b)
        clamp_flat(t, b)
        start_scatter(b)

    def grp(p, _):
        for b in range(5):
            turn(p * 5 + b, b)
        return 0

    lax.fori_loop(0, NCP // 5, grp, 0)
    for b in range(5):
        wait_scatter(b)
    plsc.subcore_barrier()

    @pl.when(sid < 15)
    def _():
        pltpu.sync_copy(acc_s.at[pl.ds(base, ST)],
                        out_hbm.at[pl.ds(clo + base, ST)])

    @pl.when(sid == 15)
    def _():
        pltpu.sync_copy(acc_s.at[pl.ds(base, ST15)],
                        out_hbm.at[pl.ds(clo + base, ST15)])


def _mm1_body(x_ref, w_ref, deg_ref, o_ref):
    dis = lax.rsqrt(1.0 + deg_ref[...])
    o_ref[...] = jnp.dot(x_ref[...], w_ref[...],
                         preferred_element_type=jnp.float32) * dis


def _mid_body(t_ref, deg_ref, b1_ref, w2_ref, o_ref):
    dis = lax.rsqrt(1.0 + deg_ref[...])
    a = jnp.maximum(t_ref[...] * dis + b1_ref[...], 0.0)
    o_ref[...] = jnp.dot(a, w2_ref[...],
                         preferred_element_type=jnp.float32) * dis


def _final_body(t_ref, deg_ref, b2_ref, bat_ref, o_ref):
    dis = lax.rsqrt(1.0 + deg_ref[...])
    h2 = t_ref[...] * dis + b2_ref[...]
    gids = lax.broadcasted_iota(jnp.int32, (R, G), 1)
    oh = (gids == bat_ref[...]).astype(jnp.float32)
    contrib = lax.dot_general(oh, h2, (((0,), (0,)), ((), ())),
                              preferred_element_type=jnp.float32)

    @pl.when(pl.program_id(0) == 0)
    def _():
        o_ref[...] = jnp.zeros_like(o_ref)

    o_ref[...] += contrib


def _mm1(x, W1, degcol):
    return pl.pallas_call(
        _mm1_body,
        grid=(N // R,),
        in_specs=[
            pl.BlockSpec((R, D), lambda i: (i, 0)),
            pl.BlockSpec((D, D), lambda i: (0, 0)),
            pl.BlockSpec((R, 1), lambda i: (i, 0)),
        ],
        out_specs=pl.BlockSpec((R, D), lambda i: (i, 0)),
        out_shape=jax.ShapeDtypeStruct((N, D), jnp.float32),
    )(x, W1, degcol)


def _mid(tmp1, degcol, b1, W2):
    return pl.pallas_call(
        _mid_body,
        grid=(N // R,),
        in_specs=[
            pl.BlockSpec((R, D), lambda i: (i, 0)),
            pl.BlockSpec((R, 1), lambda i: (i, 0)),
            pl.BlockSpec((1, D), lambda i: (0, 0)),
            pl.BlockSpec((D, D), lambda i: (0, 0)),
        ],
        out_specs=pl.BlockSpec((R, D), lambda i: (i, 0)),
        out_shape=jax.ShapeDtypeStruct((N, D), jnp.float32),
    )(tmp1, degcol, b1, W2)


def _final(tmp2, degcol, b2, bat_col):
    return pl.pallas_call(
        _final_body,
        grid=(N // R,),
        in_specs=[
            pl.BlockSpec((R, D), lambda i: (i, 0)),
            pl.BlockSpec((R, 1), lambda i: (i, 0)),
            pl.BlockSpec((1, D), lambda i: (0, 0)),
            pl.BlockSpec((R, 1), lambda i: (i, 0)),
        ],
        out_specs=pl.BlockSpec((G, D), lambda i: (0, 0)),
        out_shape=jax.ShapeDtypeStruct((G, D), jnp.float32),
        compiler_params=pltpu.CompilerParams(
            dimension_semantics=("arbitrary",)),
    )(tmp2, degcol, b2, bat_col)


def kernel(x, edge_index, batch, W1, b1, W2, b2):
    dst_r = edge_index[1].reshape(NT, NC, K)
    src_p = edge_index[0].reshape(NT, EW)
    dst_p = edge_index[1].reshape(NT, EW)
    degf = _deg(dst_r)
    degcol = degf[:, :1]
    h1s = _mm1(x, W1, degcol)
    tmp1 = _prop(h1s, src_p, dst_p)
    h2s = _mid(tmp1, degcol, b1.reshape(1, D), W2)
    tmp2 = _prop(h2s, src_p, dst_p)
    return _final(tmp2, degcol, b2.reshape(1, D), batch.reshape(N, 1))


# speedup vs baseline: 15.8786x; 1.1912x over previous
import functools

import jax
import jax.numpy as jnp
from jax import lax
from jax.experimental import pallas as pl
from jax.experimental.pallas import tpu as pltpu
from jax.experimental.pallas import tpu_sc as plsc

N = 10000
D = 128
E = 320000
G = 64
HALF = N // 2
DUMP = HALF
ACCR = HALF + 8
NT = 16
EW = E // NT
K = 80
NC = EW // K
KP = 32
NCP = EW // KP
ST = 320
ST15 = 200
R = 1000

_mesh = plsc.VectorSubcoreMesh(core_axis_name="c", subcore_axis_name="s")


def _clamp_indices(dst_v, idx_v, c, clo, k=K):
    for j in range(k // 16):
        dv = dst_v[c, pl.ds(j * 16, 16)]
        inr = (dv >= clo) & (dv < clo + HALF)
        idx_v[pl.ds(j * 16, 16)] = jnp.where(inr, dv - clo, DUMP)


NW = 32
EWD = E // NW
NCD = EWD // K
T0 = 640
T15 = 400


@functools.partial(
    pl.kernel,
    mesh=_mesh,
    out_type=jax.ShapeDtypeStruct((2, N, D), jnp.float32),
    scratch_types=[
        pltpu.VMEM((NCD, K), jnp.int32),
        pltpu.VMEM((K,), jnp.int32),
        pltpu.VMEM((K, D), jnp.float32),
        pltpu.VMEM_SHARED((N, D), jnp.float32),
    ],
)
def _deg(dst_hbm, out_hbm, dst_v, idx_v, ones_v, acc_s):
    cid = lax.axis_index("c")
    sid = lax.axis_index("s")
    wid = cid * 16 + sid
    base = sid * T0
    one = jnp.ones((16,), jnp.float32)

    def fill(i, _):
        for l in range(D // 16):
            ones_v[i, pl.ds(l * 16, 16)] = one
        return 0

    def zfill(i, _):
        for l in range(D // 16):
            ones_v[i, pl.ds(l * 16, 16)] = jnp.zeros((16,), jnp.float32)
        return 0

    lax.fori_loop(0, K, zfill, 0)

    @pl.when(sid < 15)
    def _():
        for j in range(T0 // K):
            pltpu.sync_copy(ones_v, acc_s.at[pl.ds(base + j * K, K)])

    @pl.when(sid == 15)
    def _():
        for j in range(T15 // K):
            pltpu.sync_copy(ones_v, acc_s.at[pl.ds(base + j * K, K)])

    lax.fori_loop(0, K, fill, 0)
    pltpu.sync_copy(dst_hbm.at[wid], dst_v)
    plsc.subcore_barrier()

    def step(c, _):
        for j in range(K // 16):
            idx_v[pl.ds(j * 16, 16)] = dst_v[c, pl.ds(j * 16, 16)]
        pltpu.sync_copy(ones_v, acc_s.at[idx_v], add=True)
        return 0

    lax.fori_loop(0, NCD, step, 0)
    plsc.subcore_barrier()

    @pl.when(sid < 15)
    def _():
        pltpu.sync_copy(acc_s.at[pl.ds(base, T0)],
                        out_hbm.at[cid, pl.ds(base, T0)])

    @pl.when(sid == 15)
    def _():
        pltpu.sync_copy(acc_s.at[pl.ds(base, T15)],
                        out_hbm.at[cid, pl.ds(base, T15)])


@functools.partial(
    pl.kernel,
    mesh=_mesh,
    out_type=jax.ShapeDtypeStruct((N, D), jnp.float32),
    scratch_types=[
        pltpu.VMEM((EW,), jnp.int32),
        pltpu.VMEM((EW,), jnp.int32),
        pltpu.VMEM((5, KP), jnp.int32),
        pltpu.VMEM((KP,), jnp.int32),
        pltpu.VMEM((KP,), jnp.int32),
        pltpu.VMEM((KP,), jnp.int32),
        pltpu.VMEM((KP,), jnp.int32),
        pltpu.VMEM((KP,), jnp.int32),
        pltpu.VMEM((5, KP, D), jnp.float32),
        pltpu.SemaphoreType.DMA((5,)),
        pltpu.SemaphoreType.DMA((5,)),
        pltpu.VMEM_SHARED((ACCR, D), jnp.float32),
    ],
)
def _prop(hs_hbm, src_hbm, dst_hbm, out_hbm,
          src_v, dst_v, gidx_v, ix0, ix1, ix2, ix3, ix4,
          rows_v, gsem, ssem, acc_s):
    cid = lax.axis_index("c")
    sid = lax.axis_index("s")
    clo = cid * HALF
    base = sid * ST

    @pl.when(sid < 15)
    def _():
        pltpu.sync_copy(hs_hbm.at[pl.ds(clo + base, ST)],
                        acc_s.at[pl.ds(base, ST)])

    @pl.when(sid == 15)
    def _():
        pltpu.sync_copy(hs_hbm.at[pl.ds(clo + base, ST15)],
                        acc_s.at[pl.ds(base, ST15)])

    pltpu.sync_copy(src_hbm.at[sid], src_v)
    pltpu.sync_copy(dst_hbm.at[sid], dst_v)
    plsc.subcore_barrier()

    ix = (ix0, ix1, ix2, ix3, ix4)

    def build_gidx(c, b):
        for j in range(KP // 16):
            gidx_v[b, pl.ds(j * 16, 16)] = src_v[pl.ds(c * KP + j * 16, 16)]

    def clamp_flat(c, b):
        for j in range(KP // 16):
            dv = dst_v[pl.ds(c * KP + j * 16, 16)]
            inr = (dv >= clo) & (dv < clo + HALF)
            ix[b][pl.ds(j * 16, 16)] = jnp.where(inr, dv - clo, DUMP)

    def start_gather(b):
        pltpu.async_copy(hs_hbm.at[gidx_v.at[b]], rows_v.at[b], gsem.at[b])

    def wait_gather(b):
        pltpu.make_async_copy(hs_hbm.at[gidx_v.at[b]],
                              rows_v.at[b], gsem.at[b]).wait()

    def start_scatter(b):
        pltpu.make_async_copy(rows_v.at[b], acc_s.at[ix[b]],
                              ssem.at[b]).start(add=True)

    def wait_scatter(b):
        pltpu.make_async_copy(rows_v.at[b], acc_s.at[ix[b]],
                              ssem.at[b]).wait()

    for b in range(3):
        build_gidx(jnp.int32(b), b)
        start_gather(b)

    def turn(t, b):
        bg = (b + 3) % 5

        @pl.when(t + 3 < NCP)
        def _():
            @pl.when(t >= 2)
            def _():
                wait_scatter(bg)

            build_gidx(t + 3, bg)
            start_gather(bg)

        wait_gather(b)
        clamp_flat(t, b)
        start_scatter(b)

    def grp(p, _):
        for b in range(5):
            turn(p * 5 + b, b)
        return 0

    lax.fori_loop(0, NCP // 5, grp, 0)
    for b in range(5):
        wait_scatter(b)
    plsc.subcore_barrier()

    @pl.when(sid < 15)
    def _():
        pltpu.sync_copy(acc_s.at[pl.ds(base, ST)],
                        out_hbm.at[pl.ds(clo + base, ST)])

    @pl.when(sid == 15)
    def _():
        pltpu.sync_copy(acc_s.at[pl.ds(base, ST15)],
                        out_hbm.at[pl.ds(clo + base, ST15)])


def _mm1_body(x_ref, w_ref, d0_ref, d1_ref, o_ref):
    dis = lax.rsqrt(1.0 + d0_ref[...] + d1_ref[...])
    o_ref[...] = jnp.dot(x_ref[...], w_ref[...],
                         preferred_element_type=jnp.float32) * dis


def _mid_body(t_ref, d0_ref, d1_ref, b1_ref, w2_ref, o_ref):
    dis = lax.rsqrt(1.0 + d0_ref[...] + d1_ref[...])
    a = jnp.maximum(t_ref[...] * dis + b1_ref[...], 0.0)
    o_ref[...] = jnp.dot(a, w2_ref[...],
                         preferred_element_type=jnp.float32) * dis


def _final_body(t_ref, d0_ref, d1_ref, b2_ref, bat_ref, o_ref):
    dis = lax.rsqrt(1.0 + d0_ref[...] + d1_ref[...])
    h2 = t_ref[...] * dis + b2_ref[...]
    gids = lax.broadcasted_iota(jnp.int32, (R, G), 1)
    oh = (gids == bat_ref[...]).astype(jnp.float32)
    contrib = lax.dot_general(oh, h2, (((0,), (0,)), ((), ())),
                              preferred_element_type=jnp.float32)

    @pl.when(pl.program_id(0) == 0)
    def _():
        o_ref[...] = jnp.zeros_like(o_ref)

    o_ref[...] += contrib


def _mm1(x, W1, d0, d1):
    return pl.pallas_call(
        _mm1_body,
        grid=(N // R,),
        in_specs=[
            pl.BlockSpec((R, D), lambda i: (i, 0)),
            pl.BlockSpec((D, D), lambda i: (0, 0)),
            pl.BlockSpec((R, 1), lambda i: (i, 0)),
            pl.BlockSpec((R, 1), lambda i: (i, 0)),
        ],
        out_specs=pl.BlockSpec((R, D), lambda i: (i, 0)),
        out_shape=jax.ShapeDtypeStruct((N, D), jnp.float32),
    )(x, W1, d0, d1)


def _mid(tmp1, d0, d1, b1, W2):
    return pl.pallas_call(
        _mid_body,
        grid=(N // R,),
        in_specs=[
            pl.BlockSpec((R, D), lambda i: (i, 0)),
            pl.BlockSpec((R, 1), lambda i: (i, 0)),
            pl.BlockSpec((R, 1), lambda i: (i, 0)),
            pl.BlockSpec((1, D), lambda i: (0, 0)),
            pl.BlockSpec((D, D), lambda i: (0, 0)),
        ],
        out_specs=pl.BlockSpec((R, D), lambda i: (i, 0)),
        out_shape=jax.ShapeDtypeStruct((N, D), jnp.float32),
    )(tmp1, d0, d1, b1, W2)


def _final(tmp2, d0, d1, b2, bat_col):
    return pl.pallas_call(
        _final_body,
        grid=(N // R,),
        in_specs=[
            pl.BlockSpec((R, D), lambda i: (i, 0)),
            pl.BlockSpec((R, 1), lambda i: (i, 0)),
            pl.BlockSpec((R, 1), lambda i: (i, 0)),
            pl.BlockSpec((1, D), lambda i: (0, 0)),
            pl.BlockSpec((R, 1), lambda i: (i, 0)),
        ],
        out_specs=pl.BlockSpec((G, D), lambda i: (0, 0)),
        out_shape=jax.ShapeDtypeStruct((G, D), jnp.float32),
        compiler_params=pltpu.CompilerParams(
            dimension_semantics=("arbitrary",)),
    )(tmp2, d0, d1, b2, bat_col)


def kernel(x, edge_index, batch, W1, b1, W2, b2):
    dst_d = edge_index[1].reshape(NW, NCD, K)
    src_p = edge_index[0].reshape(NT, EW)
    dst_p = edge_index[1].reshape(NT, EW)
    degf = _deg(dst_d)
    d0 = degf[0, :, :1]
    d1 = degf[1, :, :1]
    h1s = _mm1(x, W1, d0, d1)
    tmp1 = _prop(h1s, src_p, dst_p)
    h2s = _mid(tmp1, d0, d1, b1.reshape(1, D), W2)
    tmp2 = _prop(h2s, src_p, dst_p)
    return _final(tmp2, d0, d1, b2.reshape(1, D), batch.reshape(N, 1))
